# baseline (speedup 1.0000x reference)
import functools

import jax
import jax.numpy as jnp
from jax import lax
from jax.experimental import pallas as pl
from jax.experimental.pallas import tpu as pltpu
from jax.experimental.pallas import tpu_sc as plsc

N = 10000
E = 320000
D = 128
R = 8
NB = 4

NC = 2
NS = 16
NW = NC * NS
K = 128
CPW = 80
EPAD = NW * CPW * K
NP = 10240
CH = 512
NCH = NP // CH
CPS = NCH // NC
ACCR = CH * R
CNTS = NP * R
RECW = 12928

_mesh = plsc.VectorSubcoreMesh(core_axis_name="c", subcore_axis_name="s")
_sc_params = pltpu.CompilerParams(needs_layout_passes=False,
                                  use_tc_tiling_on_sc=False)

_PERM = tuple(
    [32 * (p // 16) + (p % 16) for p in range(D // 2)]
    + [32 * (p // 16) + 16 + (p % 16) for p in range(D // 2)])
_PINV = tuple(_PERM.index(d) for d in range(D))

_DUMMY = N * 4096


@functools.partial(
    pl.kernel,
    out_type=jax.ShapeDtypeStruct((CNTS,), jnp.float32),
    mesh=_mesh,
    compiler_params=_sc_params,
    scratch_types=[
        pltpu.VMEM((CPW, K), jnp.int32),
        pltpu.VMEM((CNTS // NS,), jnp.float32),
        pltpu.VMEM((K,), jnp.float32),
        pltpu.VMEM_SHARED((CNTS,), jnp.float32),
    ],
)
def _sc_prep(seg32, inv_out, segb, strip, ones, cnt):
    c = lax.axis_index("c")
    s = lax.axis_index("s")
    per = CNTS // NS

    def _fill(i, _):
        strip[pl.ds(i * 16, 16)] = jnp.zeros((16,), jnp.float32)
        return 0
    lax.fori_loop(0, per // 16, _fill, 0)

    def _fill1(i, _):
        ones[pl.ds(i * 16, 16)] = jnp.ones((16,), jnp.float32)
        return 0
    lax.fori_loop(0, K // 16, _fill1, 0)

    pltpu.sync_copy(strip, cnt.at[pl.ds(s * per, per)])
    plsc.subcore_barrier()

    def _count_w(k, _):
        pltpu.sync_copy(seg32.at[2 * s + k], segb)

        def _count(i, _):
            pltpu.sync_copy(ones, cnt.at[segb.at[i]], add=True)
            return 0
        lax.fori_loop(0, CPW, _count, 0)
        return 0
    lax.fori_loop(0, 2, _count_w, 0)
    plsc.subcore_barrier()

    pltpu.sync_copy(cnt.at[pl.ds(s * per, per)], strip)

    def _inv(i, _):
        v = strip[pl.ds(i * 16, 16)]
        strip[pl.ds(i * 16, 16)] = 1.0 / jnp.maximum(v, 1.0)
        return 0
    lax.fori_loop(0, per // 16, _inv, 0)

    @pl.when(c == 0)
    def _():
        pltpu.sync_copy(strip, inv_out.at[pl.ds(s * per, per)])


@functools.partial(
    pl.kernel,
    out_type=(
        jax.ShapeDtypeStruct((NW, RECW), jnp.int32),
        jax.ShapeDtypeStruct((NW, 48), jnp.int32),
        jax.ShapeDtypeStruct((NW, 48), jnp.int32),
    ),
    mesh=_mesh,
    compiler_params=_sc_params,
    scratch_types=[
        pltpu.VMEM((CPW, K), jnp.int32),
        pltpu.VMEM((CPW, K), jnp.int32),
        pltpu.VMEM((RECW,), jnp.int32),
        pltpu.VMEM((48,), jnp.int32),
        pltpu.VMEM((48,), jnp.int32),
    ],
)
def _sc_bucket(rec32, bkt32, recs_out, nch_out, off_out,
               recb, bktb, outb, nchb, offb):
    c = lax.axis_index("c")
    s = lax.axis_index("s")
    w = c * NS + s
    pltpu.sync_copy(rec32.at[w], recb)
    pltpu.sync_copy(bkt32.at[w], bktb)

    lane0 = lax.iota(jnp.int32, 16) == 0
    dummy = jnp.full((16,), _DUMMY, jnp.int32)

    cur = jnp.int32(0)
    for cc in range(NCH):
        start = cur
        plsc.store_scatter(offb, [jnp.full((16,), cc, jnp.int32)],
                           jnp.broadcast_to(start, (16,)), mask=lane0)

        def _scan(v, cur, cc=cc):
            q = v // 8
            l = v % 8
            m = bktb[q, pl.ds(l * 16, 16)] == cc
            rv = recb[q, pl.ds(l * 16, 16)]
            plsc.store_compressed(outb.at[pl.ds(cur, 16)], rv, mask=m)
            return cur + jnp.max(plsc.all_reduce_population_count(m))
        cur = lax.fori_loop(0, CPW * 8, _scan, cur)

        tot = cur - start
        n128 = (tot + 127) // 128
        new_cur = start + n128 * 128
        t16 = (new_cur - cur + 15) // 16

        def _pad(t, cur_):
            outb[pl.ds(cur_ + t * 16, 16)] = dummy
            return cur_
        lax.fori_loop(0, t16, _pad, cur)

        plsc.store_scatter(nchb, [jnp.full((16,), cc, jnp.int32)],
                           jnp.broadcast_to(n128, (16,)), mask=lane0)
        cur = new_cur

    pltpu.sync_copy(outb, recs_out.at[w])
    pltpu.sync_copy(nchb, nch_out.at[w])
    pltpu.sync_copy(offb, off_out.at[w])


@functools.partial(
    pl.kernel,
    out_type=jax.ShapeDtypeStruct((NCH * ACCR, D), jnp.float32),
    mesh=_mesh,
    compiler_params=_sc_params,
    scratch_types=[
        pltpu.VMEM((1, K), jnp.int32),
        pltpu.VMEM((1, K), jnp.int32),
        pltpu.VMEM((1, K), jnp.int32),
        pltpu.VMEM((K, D // 2), jnp.int32),
        pltpu.VMEM((K, D), jnp.float32),
        pltpu.VMEM((16, D), jnp.float32),
        pltpu.VMEM((NW, 32), jnp.int32),
        pltpu.VMEM((NW, 32), jnp.int32),
        pltpu.VMEM_SHARED((NP, D // 2), jnp.int32),
        pltpu.VMEM_SHARED((ACCR, D), jnp.float32),
    ],
)
def _sc_seg(xp, recs, nch, off, s_out, rbuf, gbuf, dbuf, rows, rowsf,
            zblk, nchv, offv, xsp, acc):
    c = lax.axis_index("c")
    s = lax.axis_index("s")

    def _fill(i, _):
        zblk[i // 8, pl.ds((i % 8) * 16, 16)] = jnp.zeros((16,), jnp.float32)
        return 0
    lax.fori_loop(0, 16 * 8, _fill, 0)

    pltpu.sync_copy(xp.at[pl.ds(s * (NP // NS), NP // NS)],
                    xsp.at[pl.ds(s * (NP // NS), NP // NS)])
    pltpu.sync_copy(nch, nchv)
    pltpu.sync_copy(off, offv)
    plsc.subcore_barrier()

    rpt = ACCR // NS

    def _chunk(m, _):
        cc = c * CPS + m

        def _zero(i, _):
            pltpu.sync_copy(zblk, acc.at[pl.ds(s * rpt + i * 16, 16)])
            return 0
        lax.fori_loop(0, rpt // 16, _zero, 0)
        plsc.subcore_barrier()

        iota16 = lax.iota(jnp.int32, 16)
        for k2 in range(2):
            wsrc = 2 * s + k2
            ngrp = jnp.max(jnp.where(iota16 == m,
                                     nchv[wsrc, pl.ds(c * 16, 16)], 0))
            base = pl.multiple_of(
                jnp.max(jnp.where(iota16 == m,
                                  offv[wsrc, pl.ds(c * 16, 16)], 0)), K)

            def _grp(jj, _):
                pltpu.sync_copy(recs.at[wsrc, pl.ds(base + jj * K, K)],
                                rbuf.at[0])
                for l in range(K // 16):
                    v = rbuf[0, pl.ds(l * 16, 16)]
                    gbuf[0, pl.ds(l * 16, 16)] = v >> 12
                    dbuf[0, pl.ds(l * 16, 16)] = v & jnp.full(
                        (16,), 4095, jnp.int32)
                pltpu.sync_copy(xsp.at[gbuf.at[0]], rows)

                def _unp(e, _):
                    for q in range(D // 32):
                        vv = rows[e, pl.ds(q * 16, 16)]
                        rowsf[e, pl.ds(q * 32, 16)] = plsc.bitcast(
                            vv << 16, jnp.float32)
                        rowsf[e, pl.ds(q * 32 + 16, 16)] = plsc.bitcast(
                            vv & jnp.full((16,), -65536, jnp.int32),
                            jnp.float32)
                    return 0
                lax.fori_loop(0, K, _unp, 0)

                pltpu.sync_copy(rowsf, acc.at[dbuf.at[0]], add=True)
                return 0
            lax.fori_loop(0, ngrp, _grp, 0)
        plsc.subcore_barrier()

        pltpu.sync_copy(acc.at[pl.ds(s * rpt, rpt)],
                        s_out.at[pl.ds(cc * ACCR + s * rpt, rpt)])
        plsc.subcore_barrier()
        return 0
    lax.fori_loop(0, CPS, _chunk, 0)


def _round_bf16_bits(x):
    b = lax.bitcast_convert_type(x, jnp.int32)
    r = b + 0x7FFF + ((b >> 16) & 1)
    return (r >> 16) & 0xFFFF


def _pack_body(x_ref, out_ref):
    res = x_ref[...]
    lo = _round_bf16_bits(res[:, :D // 2])
    hi = _round_bf16_bits(res[:, D // 2:])
    out_ref[...] = lo | (hi << 16)


def _tc_pack(h):
    return pl.pallas_call(
        _pack_body,
        grid=(NP // 1024,),
        in_specs=[pl.BlockSpec((1024, D), lambda j: (j, 0))],
        out_specs=pl.BlockSpec((1024, D // 2), lambda j: (j, 0)),
        out_shape=jax.ShapeDtypeStruct((NP, D // 2), jnp.int32),
    )(h)


def _comb_body(relu, s_ref, inv_ref, x_ref, wall_ref, root_ref, b_ref,
               out_ref):
    s3 = s_ref[...].reshape(CH, R, D)
    mean = (s3 * inv_ref[...][:, :, None]).reshape(CH, R * D)
    agg = lax.dot_general(
        mean, wall_ref[...], (((1,), (0,)), ((), ())),
        precision=lax.Precision.HIGHEST, preferred_element_type=jnp.float32)
    xroot = lax.dot_general(
        x_ref[...], root_ref[...], (((1,), (0,)), ((), ())),
        precision=lax.Precision.HIGHEST, preferred_element_type=jnp.float32)
    res = agg + xroot + b_ref[0]
    if relu:
        res = jnp.maximum(res, 0.0)
    rid = pl.program_id(0) * CH + lax.broadcasted_iota(jnp.int32, (CH, 1), 0)
    out_ref[...] = jnp.where(rid < N, res, 0.0)


def _tc_combine(s_tab, inv8, h, wall, root, bias2d, relu):
    return pl.pallas_call(
        functools.partial(_comb_body, relu),
        grid=(NCH,),
        in_specs=[
            pl.BlockSpec((ACCR, D), lambda j: (j, 0)),
            pl.BlockSpec((CH, R), lambda j: (j, 0)),
            pl.BlockSpec((CH, D), lambda j: (j, 0)),
            pl.BlockSpec((R * D, D), lambda j: (0, 0)),
            pl.BlockSpec((D, D), lambda j: (0, 0)),
            pl.BlockSpec((8, D), lambda j: (0, 0)),
        ],
        out_specs=pl.BlockSpec((CH, D), lambda j: (j, 0)),
        out_shape=jax.ShapeDtypeStruct((NP, D), jnp.float32),
    )(s_tab, inv8, h, wall, root, bias2d)


def _layer(h, bases, comp, root, bias, recs, nch, off, inv8, relu):
    wall = jnp.einsum('rb,bio->rio', comp, bases)[
        :, jnp.array(_PINV, jnp.int32), :].reshape(R * D, D)
    xp = _tc_pack(h)
    s_tab = _sc_seg(xp, recs, nch, off)
    bias2d = jnp.broadcast_to(bias, (8, D))
    return _tc_combine(s_tab, inv8, h, wall, root, bias2d, relu)


def kernel(x, t, obj_cond, edge_cond, relation_cond, bases1, comp1, root1,
           bias1, bases2, comp2, root2, bias2):
    h = jnp.concatenate(
        [x.reshape(N, D), jnp.zeros((NP - N, D), jnp.float32)])
    src = edge_cond[0]
    dst = edge_cond[1]
    rel = relation_cond
    pad = EPAD - E
    src_p = jnp.concatenate([src, jnp.zeros((pad,), jnp.int32)])
    dst_p = jnp.concatenate([dst, jnp.full((pad,), N, jnp.int32)])
    rel_p = jnp.concatenate([rel, jnp.zeros((pad,), jnp.int32)])
    seg = dst_p * R + rel_p
    rec = src_p * 4096 + (dst_p & 511) * R + rel_p
    bkt = dst_p >> 9

    inv = _sc_prep(seg.reshape(NW, CPW, K))
    recs, nch, off = _sc_bucket(rec.reshape(NW, CPW, K),
                                bkt.reshape(NW, CPW, K))
    nch_r = (jnp.zeros((NW, 32), jnp.int32)
             .at[:, :CPS].set(nch[:, :CPS])
             .at[:, 16:16 + CPS].set(nch[:, CPS:NCH]))
    off_r = (jnp.zeros((NW, 32), jnp.int32)
             .at[:, :CPS].set(off[:, :CPS])
             .at[:, 16:16 + CPS].set(off[:, CPS:NCH]))
    inv8 = inv.reshape(NP, R)

    h1 = _layer(h, bases1, comp1, root1, bias1, recs, nch_r, off_r, inv8,
                True)
    h2 = _layer(h1, bases2, comp2, root2, bias2, recs, nch_r, off_r, inv8,
                False)
    return h2[:N].reshape(1, N, D)


# speedup vs baseline: 1.6428x; 1.6428x over previous
import functools

import jax
import jax.numpy as jnp
from jax import lax
from jax.experimental import pallas as pl
from jax.experimental.pallas import tpu as pltpu
from jax.experimental.pallas import tpu_sc as plsc

N = 10000
E = 320000
D = 128
R = 8
NB = 4

NC = 2
NS = 16
NW = NC * NS
K = 128
CPW = 80
GRP = 16
EPAD = NW * CPW * K
CP16 = EPAD // NS // K
CNTS = 80128
NACC = 10240
MMB = 1000

_mesh = plsc.VectorSubcoreMesh(core_axis_name="c", subcore_axis_name="s")
_sc_params = pltpu.CompilerParams(needs_layout_passes=False,
                                  use_tc_tiling_on_sc=False)


@functools.partial(
    pl.kernel,
    out_type=jax.ShapeDtypeStruct((NW, CPW, K), jnp.float32),
    mesh=_mesh,
    compiler_params=_sc_params,
    scratch_types=[
        pltpu.VMEM((CPW, K), jnp.int32),
        pltpu.VMEM((CPW, K), jnp.float32),
        pltpu.VMEM((CNTS // NS,), jnp.float32),
        pltpu.VMEM((K,), jnp.float32),
        pltpu.VMEM_SHARED((CNTS,), jnp.float32),
    ],
)
def _sc_prep(seg32, scale_out, segb, sbuf, strip, ones, cnt):
    c = lax.axis_index("c")
    s = lax.axis_index("s")
    w = c * NS + s
    per = CNTS // NS

    def _fill(i, _):
        strip[pl.ds(i * 16, 16)] = jnp.zeros((16,), jnp.float32)
        return 0
    lax.fori_loop(0, per // 16, _fill, 0)

    def _fill1(i, _):
        ones[pl.ds(i * 16, 16)] = jnp.ones((16,), jnp.float32)
        return 0
    lax.fori_loop(0, K // 16, _fill1, 0)

    pltpu.sync_copy(strip, cnt.at[pl.ds(s * per, per)])
    plsc.subcore_barrier()

    def _count_w(k, _):
        pltpu.sync_copy(seg32.at[2 * s + k], segb)

        def _count(i, _):
            pltpu.sync_copy(ones, cnt.at[segb.at[i]], add=True)
            return 0
        lax.fori_loop(0, CPW, _count, 0)
        return 0
    lax.fori_loop(0, 2, _count_w, 0)
    plsc.subcore_barrier()

    pltpu.sync_copy(cnt.at[pl.ds(s * per, per)], strip)

    def _inv(i, _):
        v = strip[pl.ds(i * 16, 16)]
        strip[pl.ds(i * 16, 16)] = 1.0 / jnp.maximum(v, 1.0)
        return 0
    lax.fori_loop(0, per // 16, _inv, 0)
    pltpu.sync_copy(strip, cnt.at[pl.ds(s * per, per)])
    plsc.subcore_barrier()

    pltpu.sync_copy(seg32.at[w], segb)

    def _gath(i, _):
        pltpu.sync_copy(cnt.at[segb.at[i]], sbuf.at[i])
        return 0
    lax.fori_loop(0, CPW, _gath, 0)
    pltpu.sync_copy(sbuf, scale_out.at[w])


@functools.partial(
    pl.kernel,
    out_type=jax.ShapeDtypeStruct((NC, NACC, D), jnp.float32),
    mesh=_mesh,
    compiler_params=_sc_params,
    scratch_types=[
        pltpu.VMEM((GRP, K), jnp.int32),
        pltpu.VMEM((GRP, K), jnp.int32),
        pltpu.VMEM((GRP, K), jnp.float32),
        pltpu.VMEM((K, D // 2), jnp.int32),
        pltpu.VMEM((K, D // 2), jnp.int32),
        pltpu.VMEM((K, D), jnp.float32),
        pltpu.VMEM((16, D), jnp.float32),
        pltpu.VMEM_SHARED((NACC, D), jnp.float32),
        pltpu.SemaphoreType.DMA,
        pltpu.SemaphoreType.DMA,
    ],
)
def _sc_scatter(table, gidx32, dst32, scale32, parts, gbuf, dbuf, sbuf,
                rows0, rows1, rowsf, zblk, acc, sem0, sem1):
    c = lax.axis_index("c")
    s = lax.axis_index("s")
    w = c * NS + s
    rpt = NACC // NS

    def _fill(i, _):
        zblk[i // 8, pl.ds((i % 8) * 16, 16)] = jnp.zeros((16,), jnp.float32)
        return 0
    lax.fori_loop(0, 16 * 8, _fill, 0)

    def _zero(i, _):
        pltpu.sync_copy(zblk, acc.at[pl.ds(s * rpt + i * 16, 16)])
        return 0
    lax.fori_loop(0, rpt // 16, _zero, 0)
    plsc.subcore_barrier()

    def _process(i, rows):
        def _scale_edge(e, _):
            sv = plsc.load_gather(sbuf, [jnp.full((16,), i, jnp.int32),
                                         jnp.full((16,), e, jnp.int32)])
            for q in range(D // 32):
                v = rows[e, pl.ds(q * 16, 16)]
                a = plsc.bitcast(v << 16, jnp.float32)
                b = plsc.bitcast(v & jnp.full((16,), -65536, jnp.int32),
                                 jnp.float32)
                rowsf[e, pl.ds(q * 32, 16)] = a * sv
                rowsf[e, pl.ds(q * 32 + 16, 16)] = b * sv
            return 0
        lax.fori_loop(0, K, _scale_edge, 0)
        pltpu.sync_copy(rowsf, acc.at[dbuf.at[i]], add=True)

    def _group(g, _):
        pltpu.sync_copy(gidx32.at[w, pl.ds(g * GRP, GRP)], gbuf)
        pltpu.sync_copy(dst32.at[w, pl.ds(g * GRP, GRP)], dbuf)
        pltpu.sync_copy(scale32.at[w, pl.ds(g * GRP, GRP)], sbuf)

        pltpu.async_copy(table.at[gbuf.at[0]], rows0, sem0)

        def _pair(j, _):
            i0 = 2 * j
            i1 = 2 * j + 1
            pltpu.async_copy(table.at[gbuf.at[i1]], rows1, sem1)
            pltpu.make_async_copy(table.at[gbuf.at[i0]], rows0, sem0).wait()
            _process(i0, rows0)

            @pl.when(j < GRP // 2 - 1)
            def _():
                pltpu.async_copy(table.at[gbuf.at[i0 + 2]], rows0, sem0)
            pltpu.make_async_copy(table.at[gbuf.at[i1]], rows1, sem1).wait()
            _process(i1, rows1)
            return 0
        lax.fori_loop(0, GRP // 2, _pair, 0)
        return 0
    lax.fori_loop(0, CPW // GRP, _group, 0)
    plsc.subcore_barrier()

    pltpu.sync_copy(acc.at[pl.ds(s * rpt, rpt)],
                    parts.at[c, pl.ds(s * rpt, rpt)])


def _round_bf16_bits(x):
    b = lax.bitcast_convert_type(x, jnp.int32)
    r = b + 0x7FFF + ((b >> 16) & 1)
    return (r >> 16) & 0xFFFF


def _tab_body(x_ref, b_ref, c_ref, out_ref):
    r = pl.program_id(0)
    wgt = jnp.zeros((D, D), jnp.float32)
    for b in range(NB):
        wgt = wgt + c_ref[r, b] * b_ref[b]
    res = lax.dot_general(
        x_ref[...], wgt, (((1,), (0,)), ((), ())),
        precision=lax.Precision.HIGHEST, preferred_element_type=jnp.float32)
    lo = _round_bf16_bits(res[:, :D // 2])
    hi = _round_bf16_bits(res[:, D // 2:])
    out_ref[0] = lo | (hi << 16)


def _tc_tables(h, bases_p, comp):
    return pl.pallas_call(
        _tab_body,
        grid=(R, N // MMB),
        in_specs=[
            pl.BlockSpec((MMB, D), lambda r, j: (j, 0)),
            pl.BlockSpec((NB, D, D), lambda r, j: (0, 0, 0)),
            pl.BlockSpec(memory_space=pltpu.SMEM),
        ],
        out_specs=pl.BlockSpec((1, MMB, D // 2), lambda r, j: (r, j, 0)),
        out_shape=jax.ShapeDtypeStruct((R, N, D // 2), jnp.int32),
    )(h, bases_p, comp)


def _add_body(relu, p_ref, x_ref, r_ref, b_ref, out_ref):
    xroot = lax.dot_general(
        x_ref[...], r_ref[...], (((1,), (0,)), ((), ())),
        precision=lax.Precision.HIGHEST, preferred_element_type=jnp.float32)
    res = p_ref[0] + p_ref[1] + xroot + b_ref[0]
    if relu:
        res = jnp.maximum(res, 0.0)
    out_ref[...] = res


def _tc_combine(parts, h, root, bias2d, relu):
    return pl.pallas_call(
        functools.partial(_add_body, relu),
        grid=(N // MMB,),
        in_specs=[
            pl.BlockSpec((NC, MMB, D), lambda j: (0, j, 0)),
            pl.BlockSpec((MMB, D), lambda j: (j, 0)),
            pl.BlockSpec((D, D), lambda j: (0, 0)),
            pl.BlockSpec((8, D), lambda j: (0, 0)),
        ],
        out_specs=pl.BlockSpec((MMB, D), lambda j: (j, 0)),
        out_shape=jax.ShapeDtypeStruct((N, D), jnp.float32),
    )(parts, h, root, bias2d)


_PERM = tuple(
    [32 * (p // 16) + (p % 16) for p in range(D // 2)]
    + [32 * (p // 16) + 16 + (p % 16) for p in range(D // 2)])


def _layer(h, bases, comp, root, bias, gidx32, dst32, scale32, relu):
    bases_p = bases[:, :, jnp.array(_PERM, jnp.int32)]
    tab = _tc_tables(h, bases_p, comp)
    parts = _sc_scatter(tab.reshape(R * N, D // 2), gidx32, dst32, scale32)
    bias2d = jnp.broadcast_to(bias, (8, D))
    return _tc_combine(parts, h, root, bias2d, relu)


def kernel(x, t, obj_cond, edge_cond, relation_cond, bases1, comp1, root1,
           bias1, bases2, comp2, root2, bias2):
    h = x.reshape(N, D)
    src = edge_cond[0]
    dst = edge_cond[1]
    rel = relation_cond
    pad = EPAD - E
    src_p = jnp.concatenate([src, jnp.zeros((pad,), jnp.int32)])
    dst_p = jnp.concatenate([dst, jnp.full((pad,), N, jnp.int32)])
    rel_p = jnp.concatenate([rel, jnp.zeros((pad,), jnp.int32)])
    seg = dst_p * R + rel_p
    gidx = rel_p * N + src_p
    dst_p = jnp.minimum(dst_p, NACC - 1)

    scale32 = _sc_prep(seg.reshape(NW, CPW, K))
    gidx32 = gidx.reshape(NW, CPW, K)
    dst32 = dst_p.reshape(NW, CPW, K)

    h1 = _layer(h, bases1, comp1, root1, bias1, gidx32, dst32, scale32, True)
    h2 = _layer(h1, bases2, comp2, root2, bias2, gidx32, dst32, scale32, False)
    return h2.reshape(1, N, D)


# speedup vs baseline: 1.7363x; 1.0569x over previous
import functools

import jax
import jax.numpy as jnp
from jax import lax
from jax.experimental import pallas as pl
from jax.experimental.pallas import tpu as pltpu
from jax.experimental.pallas import tpu_sc as plsc

N = 10000
E = 320000
D = 128
R = 8
NB = 4

NC = 2
NS = 16
NW = NC * NS
K = 128
CPW = 80
GRP = 16
EPAD = NW * CPW * K
CP16 = EPAD // NS // K
CNTS = 80128
NACC = 10240
MMB = 1000

_mesh = plsc.VectorSubcoreMesh(core_axis_name="c", subcore_axis_name="s")
_sc_params = pltpu.CompilerParams(needs_layout_passes=False,
                                  use_tc_tiling_on_sc=False)


@functools.partial(
    pl.kernel,
    out_type=jax.ShapeDtypeStruct((NW, CPW, K), jnp.float32),
    mesh=_mesh,
    compiler_params=_sc_params,
    scratch_types=[
        pltpu.VMEM((CPW, K), jnp.int32),
        pltpu.VMEM((CPW, K), jnp.float32),
        pltpu.VMEM((CNTS // NS,), jnp.float32),
        pltpu.VMEM((K,), jnp.float32),
        pltpu.VMEM_SHARED((CNTS,), jnp.float32),
    ],
)
def _sc_prep(seg32, scale_out, segb, sbuf, strip, ones, cnt):
    c = lax.axis_index("c")
    s = lax.axis_index("s")
    w = c * NS + s
    per = CNTS // NS

    def _fill(i, _):
        strip[pl.ds(i * 16, 16)] = jnp.zeros((16,), jnp.float32)
        return 0
    lax.fori_loop(0, per // 16, _fill, 0)

    def _fill1(i, _):
        ones[pl.ds(i * 16, 16)] = jnp.ones((16,), jnp.float32)
        return 0
    lax.fori_loop(0, K // 16, _fill1, 0)

    pltpu.sync_copy(strip, cnt.at[pl.ds(s * per, per)])
    plsc.subcore_barrier()

    def _count_w(k, _):
        pltpu.sync_copy(seg32.at[2 * s + k], segb)

        def _count(i, _):
            pltpu.sync_copy(ones, cnt.at[segb.at[i]], add=True)
            return 0
        lax.fori_loop(0, CPW, _count, 0)
        return 0
    lax.fori_loop(0, 2, _count_w, 0)
    plsc.subcore_barrier()

    pltpu.sync_copy(cnt.at[pl.ds(s * per, per)], strip)

    def _inv(i, _):
        v = strip[pl.ds(i * 16, 16)]
        strip[pl.ds(i * 16, 16)] = 1.0 / jnp.maximum(v, 1.0)
        return 0
    lax.fori_loop(0, per // 16, _inv, 0)
    pltpu.sync_copy(strip, cnt.at[pl.ds(s * per, per)])
    plsc.subcore_barrier()

    pltpu.sync_copy(seg32.at[w], segb)

    def _gath(i, _):
        pltpu.sync_copy(cnt.at[segb.at[i]], sbuf.at[i])
        return 0
    lax.fori_loop(0, CPW, _gath, 0)
    pltpu.sync_copy(sbuf, scale_out.at[w])


@functools.partial(
    pl.kernel,
    out_type=jax.ShapeDtypeStruct((NC, NACC, D), jnp.float32),
    mesh=_mesh,
    compiler_params=_sc_params,
    scratch_types=[
        pltpu.VMEM((GRP, K), jnp.int32),
        pltpu.VMEM((GRP, K), jnp.int32),
        pltpu.VMEM((GRP, K), jnp.float32),
        pltpu.VMEM((K, D // 2), jnp.int32),
        pltpu.VMEM((K, D // 2), jnp.int32),
        pltpu.VMEM((K, D), jnp.float32),
        pltpu.VMEM((16, D), jnp.float32),
        pltpu.VMEM_SHARED((NACC, D), jnp.float32),
        pltpu.SemaphoreType.DMA,
        pltpu.SemaphoreType.DMA,
    ],
)
def _sc_scatter(table, gidx32, dst32, scale32, parts, gbuf, dbuf, sbuf,
                rows0, rows1, rowsf, zblk, acc, sem0, sem1):
    c = lax.axis_index("c")
    s = lax.axis_index("s")
    w = c * NS + s
    rpt = NACC // NS

    def _fill(i, _):
        zblk[i // 8, pl.ds((i % 8) * 16, 16)] = jnp.zeros((16,), jnp.float32)
        return 0
    lax.fori_loop(0, 16 * 8, _fill, 0)

    def _zero(i, _):
        pltpu.sync_copy(zblk, acc.at[pl.ds(s * rpt + i * 16, 16)])
        return 0
    lax.fori_loop(0, rpt // 16, _zero, 0)
    plsc.subcore_barrier()

    def _process(i, rows):
        def _scale_edge(e, _):
            sv = plsc.load_gather(sbuf, [jnp.full((16,), i, jnp.int32),
                                         jnp.full((16,), e, jnp.int32)])
            for q in range(D // 32):
                v = rows[e, pl.ds(q * 16, 16)]
                a = plsc.bitcast(v << 16, jnp.float32)
                b = plsc.bitcast(v & jnp.full((16,), -65536, jnp.int32),
                                 jnp.float32)
                rowsf[e, pl.ds(q * 32, 16)] = a * sv
                rowsf[e, pl.ds(q * 32 + 16, 16)] = b * sv
            return 0
        lax.fori_loop(0, K, _scale_edge, 0)
        pltpu.sync_copy(rowsf, acc.at[dbuf.at[i]], add=True)

    def _group(g, _):
        pltpu.sync_copy(gidx32.at[w, pl.ds(g * GRP, GRP)], gbuf)
        pltpu.sync_copy(dst32.at[w, pl.ds(g * GRP, GRP)], dbuf)
        pltpu.sync_copy(scale32.at[w, pl.ds(g * GRP, GRP)], sbuf)

        pltpu.async_copy(table.at[gbuf.at[0]], rows0, sem0)

        def _pair(j, _):
            i0 = 2 * j
            i1 = 2 * j + 1
            pltpu.async_copy(table.at[gbuf.at[i1]], rows1, sem1)
            pltpu.make_async_copy(table.at[gbuf.at[i0]], rows0, sem0).wait()
            _process(i0, rows0)

            @pl.when(j < GRP // 2 - 1)
            def _():
                pltpu.async_copy(table.at[gbuf.at[i0 + 2]], rows0, sem0)
            pltpu.make_async_copy(table.at[gbuf.at[i1]], rows1, sem1).wait()
            _process(i1, rows1)
            return 0
        lax.fori_loop(0, GRP // 2, _pair, 0)
        return 0
    lax.fori_loop(0, CPW // GRP, _group, 0)
    plsc.subcore_barrier()

    pltpu.sync_copy(acc.at[pl.ds(s * rpt, rpt)],
                    parts.at[c, pl.ds(s * rpt, rpt)])


def _round_bf16_bits(x):
    b = lax.bitcast_convert_type(x, jnp.int32)
    r = b + 0x7FFF + ((b >> 16) & 1)
    return (r >> 16) & 0xFFFF


def _tab_body(x_ref, b_ref, c_ref, out_ref):
    r = pl.program_id(0)
    wgt = jnp.zeros((D, D), jnp.float32)
    for b in range(NB):
        wgt = wgt + c_ref[r, b] * b_ref[b]
    res = lax.dot_general(
        x_ref[...], wgt, (((1,), (0,)), ((), ())),
        precision=lax.Precision.DEFAULT, preferred_element_type=jnp.float32)
    lo = _round_bf16_bits(res[:, :D // 2])
    hi = _round_bf16_bits(res[:, D // 2:])
    out_ref[0] = lo | (hi << 16)


def _tc_tables(h, bases_p, comp):
    return pl.pallas_call(
        _tab_body,
        grid=(R, N // MMB),
        in_specs=[
            pl.BlockSpec((MMB, D), lambda r, j: (j, 0)),
            pl.BlockSpec((NB, D, D), lambda r, j: (0, 0, 0)),
            pl.BlockSpec(memory_space=pltpu.SMEM),
        ],
        out_specs=pl.BlockSpec((1, MMB, D // 2), lambda r, j: (r, j, 0)),
        out_shape=jax.ShapeDtypeStruct((R, N, D // 2), jnp.int32),
    )(h, bases_p, comp)


def _add_body(relu, p_ref, x_ref, r_ref, b_ref, out_ref):
    xroot = lax.dot_general(
        x_ref[...], r_ref[...], (((1,), (0,)), ((), ())),
        precision=lax.Precision.HIGHEST, preferred_element_type=jnp.float32)
    res = p_ref[0] + p_ref[1] + xroot + b_ref[0]
    if relu:
        res = jnp.maximum(res, 0.0)
    out_ref[...] = res


def _tc_combine(parts, h, root, bias2d, relu):
    return pl.pallas_call(
        functools.partial(_add_body, relu),
        grid=(N // MMB,),
        in_specs=[
            pl.BlockSpec((NC, MMB, D), lambda j: (0, j, 0)),
            pl.BlockSpec((MMB, D), lambda j: (j, 0)),
            pl.BlockSpec((D, D), lambda j: (0, 0)),
            pl.BlockSpec((8, D), lambda j: (0, 0)),
        ],
        out_specs=pl.BlockSpec((MMB, D), lambda j: (j, 0)),
        out_shape=jax.ShapeDtypeStruct((N, D), jnp.float32),
    )(parts, h, root, bias2d)


_PERM = tuple(
    [32 * (p // 16) + (p % 16) for p in range(D // 2)]
    + [32 * (p // 16) + 16 + (p % 16) for p in range(D // 2)])


def _layer(h, bases, comp, root, bias, gidx32, dst32, scale32, relu):
    bases_p = bases[:, :, jnp.array(_PERM, jnp.int32)]
    tab = _tc_tables(h, bases_p, comp)
    parts = _sc_scatter(tab.reshape(R * N, D // 2), gidx32, dst32, scale32)
    bias2d = jnp.broadcast_to(bias, (8, D))
    return _tc_combine(parts, h, root, bias2d, relu)


def kernel(x, t, obj_cond, edge_cond, relation_cond, bases1, comp1, root1,
           bias1, bases2, comp2, root2, bias2):
    h = x.reshape(N, D)
    src = edge_cond[0]
    dst = edge_cond[1]
    rel = relation_cond
    pad = EPAD - E
    src_p = jnp.concatenate([src, jnp.zeros((pad,), jnp.int32)])
    dst_p = jnp.concatenate([dst, jnp.full((pad,), N, jnp.int32)])
    rel_p = jnp.concatenate([rel, jnp.zeros((pad,), jnp.int32)])
    seg = dst_p * R + rel_p
    gidx = rel_p * N + src_p
    dst_p = jnp.minimum(dst_p, NACC - 1)

    scale32 = _sc_prep(seg.reshape(NW, CPW, K))
    gidx32 = gidx.reshape(NW, CPW, K)
    dst32 = dst_p.reshape(NW, CPW, K)

    h1 = _layer(h, bases1, comp1, root1, bias1, gidx32, dst32, scale32, True)
    h2 = _layer(h1, bases2, comp2, root2, bias2, gidx32, dst32, scale32, False)
    return h2.reshape(1, N, D)


# speedup vs baseline: 1.7468x; 1.0060x over previous
import functools

import jax
import jax.numpy as jnp
from jax import lax
from jax.experimental import pallas as pl
from jax.experimental.pallas import tpu as pltpu
from jax.experimental.pallas import tpu_sc as plsc

N = 10000
E = 320000
D = 128
R = 8
NB = 4

NC = 2
NS = 16
NW = NC * NS
K = 128
CPW = 80
GRP = 16
EPAD = NW * CPW * K
CP16 = EPAD // NS // K
CNTS = 80128
NACC = 10240
MMB = 1000

_mesh = plsc.VectorSubcoreMesh(core_axis_name="c", subcore_axis_name="s")
_sc_params = pltpu.CompilerParams(needs_layout_passes=False,
                                  use_tc_tiling_on_sc=False)


@functools.partial(
    pl.kernel,
    out_type=jax.ShapeDtypeStruct((NW, CPW, K), jnp.float32),
    mesh=_mesh,
    compiler_params=_sc_params,
    scratch_types=[
        pltpu.VMEM((CPW, K), jnp.int32),
        pltpu.VMEM((CPW, K), jnp.float32),
        pltpu.VMEM((CNTS // NS,), jnp.float32),
        pltpu.VMEM((K,), jnp.float32),
        pltpu.VMEM_SHARED((CNTS,), jnp.float32),
    ],
)
def _sc_prep(seg32, scale_out, segb, sbuf, strip, ones, cnt):
    c = lax.axis_index("c")
    s = lax.axis_index("s")
    w = c * NS + s
    per = CNTS // NS

    def _fill(i, _):
        strip[pl.ds(i * 16, 16)] = jnp.zeros((16,), jnp.float32)
        return 0
    lax.fori_loop(0, per // 16, _fill, 0)

    def _fill1(i, _):
        ones[pl.ds(i * 16, 16)] = jnp.ones((16,), jnp.float32)
        return 0
    lax.fori_loop(0, K // 16, _fill1, 0)

    pltpu.sync_copy(strip, cnt.at[pl.ds(s * per, per)])
    plsc.subcore_barrier()

    def _count_w(k, _):
        pltpu.sync_copy(seg32.at[2 * s + k], segb)

        def _count(i, _):
            pltpu.sync_copy(ones, cnt.at[segb.at[i]], add=True)
            return 0
        lax.fori_loop(0, CPW, _count, 0)
        return 0
    lax.fori_loop(0, 2, _count_w, 0)
    plsc.subcore_barrier()

    pltpu.sync_copy(cnt.at[pl.ds(s * per, per)], strip)

    def _inv(i, _):
        v = strip[pl.ds(i * 16, 16)]
        strip[pl.ds(i * 16, 16)] = 1.0 / jnp.maximum(v, 1.0)
        return 0
    lax.fori_loop(0, per // 16, _inv, 0)
    pltpu.sync_copy(strip, cnt.at[pl.ds(s * per, per)])
    plsc.subcore_barrier()

    pltpu.sync_copy(seg32.at[w], segb)

    def _gath(i, _):
        pltpu.sync_copy(cnt.at[segb.at[i]], sbuf.at[i])
        return 0
    lax.fori_loop(0, CPW, _gath, 0)
    pltpu.sync_copy(sbuf, scale_out.at[w])


@functools.partial(
    pl.kernel,
    out_type=jax.ShapeDtypeStruct((NC, NACC, D), jnp.float32),
    mesh=_mesh,
    compiler_params=_sc_params,
    scratch_types=[
        pltpu.VMEM((GRP, K), jnp.int32),
        pltpu.VMEM((GRP, K), jnp.int32),
        pltpu.VMEM((GRP, K), jnp.float32),
        pltpu.VMEM((K, D // 2), jnp.int32),
        pltpu.VMEM((K, D // 2), jnp.int32),
        pltpu.VMEM((K, D), jnp.float32),
        pltpu.VMEM((16, D), jnp.float32),
        pltpu.VMEM_SHARED((NACC, D), jnp.float32),
        pltpu.SemaphoreType.DMA,
        pltpu.SemaphoreType.DMA,
    ],
)
def _sc_scatter(table, gidx32, dst32, scale32, parts, gbuf, dbuf, sbuf,
                rows0, rows1, rowsf, zblk, acc, sem0, sem1):
    c = lax.axis_index("c")
    s = lax.axis_index("s")
    w = c * NS + s
    rpt = NACC // NS

    def _fill(i, _):
        zblk[i // 8, pl.ds((i % 8) * 16, 16)] = jnp.zeros((16,), jnp.float32)
        return 0
    lax.fori_loop(0, 16 * 8, _fill, 0)

    def _zero(i, _):
        pltpu.sync_copy(zblk, acc.at[pl.ds(s * rpt + i * 16, 16)])
        return 0
    lax.fori_loop(0, rpt // 16, _zero, 0)
    plsc.subcore_barrier()

    def _process(i, rows):
        def _scale_edge(e, _):
            sv = plsc.load_gather(sbuf, [jnp.full((16,), i, jnp.int32),
                                         jnp.full((16,), e, jnp.int32)])
            for q in range(D // 32):
                v = rows[e, pl.ds(q * 16, 16)]
                a = plsc.bitcast(v << 16, jnp.float32)
                b = plsc.bitcast(v & jnp.full((16,), -65536, jnp.int32),
                                 jnp.float32)
                rowsf[e, pl.ds(q * 32, 16)] = a * sv
                rowsf[e, pl.ds(q * 32 + 16, 16)] = b * sv
            return 0
        lax.fori_loop(0, K, _scale_edge, 0)
        pltpu.sync_copy(rowsf, acc.at[dbuf.at[i]], add=True)

    def _group(g, _):
        pltpu.sync_copy(gidx32.at[w, pl.ds(g * GRP, GRP)], gbuf)
        pltpu.sync_copy(dst32.at[w, pl.ds(g * GRP, GRP)], dbuf)
        pltpu.sync_copy(scale32.at[w, pl.ds(g * GRP, GRP)], sbuf)

        pltpu.async_copy(table.at[gbuf.at[0]], rows0, sem0)

        def _pair(j, _):
            i0 = 2 * j
            i1 = 2 * j + 1
            pltpu.async_copy(table.at[gbuf.at[i1]], rows1, sem1)
            pltpu.make_async_copy(table.at[gbuf.at[i0]], rows0, sem0).wait()
            _process(i0, rows0)

            @pl.when(j < GRP // 2 - 1)
            def _():
                pltpu.async_copy(table.at[gbuf.at[i0 + 2]], rows0, sem0)
            pltpu.make_async_copy(table.at[gbuf.at[i1]], rows1, sem1).wait()
            _process(i1, rows1)
            return 0
        lax.fori_loop(0, GRP // 2, _pair, 0)
        return 0
    lax.fori_loop(0, CPW // GRP, _group, 0)
    plsc.subcore_barrier()

    pltpu.sync_copy(acc.at[pl.ds(s * rpt, rpt)],
                    parts.at[c, pl.ds(s * rpt, rpt)])


def _round_bf16_bits(x):
    b = lax.bitcast_convert_type(x, jnp.int32)
    r = b + 0x7FFF + ((b >> 16) & 1)
    return (r >> 16) & 0xFFFF


def _tab_body(x_ref, b_ref, c_ref, out_ref):
    r = pl.program_id(0)
    wgt = jnp.zeros((D, D), jnp.float32)
    for b in range(NB):
        wgt = wgt + c_ref[r, b] * b_ref[b]
    res = lax.dot_general(
        x_ref[...], wgt, (((1,), (0,)), ((), ())),
        precision=lax.Precision.DEFAULT, preferred_element_type=jnp.float32)
    lo = _round_bf16_bits(res[:, :D // 2])
    hi = _round_bf16_bits(res[:, D // 2:])
    out_ref[0] = lo | (hi << 16)


def _tc_tables(h, bases_p, comp):
    return pl.pallas_call(
        _tab_body,
        grid=(R, N // MMB),
        in_specs=[
            pl.BlockSpec((MMB, D), lambda r, j: (j, 0)),
            pl.BlockSpec((NB, D, D), lambda r, j: (0, 0, 0)),
            pl.BlockSpec(memory_space=pltpu.SMEM),
        ],
        out_specs=pl.BlockSpec((1, MMB, D // 2), lambda r, j: (r, j, 0)),
        out_shape=jax.ShapeDtypeStruct((R, N, D // 2), jnp.int32),
    )(h, bases_p, comp)


def _add_body(relu, p_ref, x_ref, r_ref, b_ref, out_ref):
    xroot = lax.dot_general(
        x_ref[...], r_ref[...], (((1,), (0,)), ((), ())),
        precision=lax.Precision.DEFAULT, preferred_element_type=jnp.float32)
    res = p_ref[0] + p_ref[1] + xroot + b_ref[0]
    if relu:
        res = jnp.maximum(res, 0.0)
    out_ref[...] = res


def _tc_combine(parts, h, root, bias2d, relu):
    return pl.pallas_call(
        functools.partial(_add_body, relu),
        grid=(N // MMB,),
        in_specs=[
            pl.BlockSpec((NC, MMB, D), lambda j: (0, j, 0)),
            pl.BlockSpec((MMB, D), lambda j: (j, 0)),
            pl.BlockSpec((D, D), lambda j: (0, 0)),
            pl.BlockSpec((8, D), lambda j: (0, 0)),
        ],
        out_specs=pl.BlockSpec((MMB, D), lambda j: (j, 0)),
        out_shape=jax.ShapeDtypeStruct((N, D), jnp.float32),
    )(parts, h, root, bias2d)


_PERM = tuple(
    [32 * (p // 16) + (p % 16) for p in range(D // 2)]
    + [32 * (p // 16) + 16 + (p % 16) for p in range(D // 2)])


def _layer(h, bases, comp, root, bias, gidx32, dst32, scale32, relu):
    bases_p = bases[:, :, jnp.array(_PERM, jnp.int32)]
    tab = _tc_tables(h, bases_p, comp)
    parts = _sc_scatter(tab.reshape(R * N, D // 2), gidx32, dst32, scale32)
    bias2d = jnp.broadcast_to(bias, (8, D))
    return _tc_combine(parts, h, root, bias2d, relu)


def kernel(x, t, obj_cond, edge_cond, relation_cond, bases1, comp1, root1,
           bias1, bases2, comp2, root2, bias2):
    h = x.reshape(N, D)
    src = edge_cond[0]
    dst = edge_cond[1]
    rel = relation_cond
    pad = EPAD - E
    src_p = jnp.concatenate([src, jnp.zeros((pad,), jnp.int32)])
    dst_p = jnp.concatenate([dst, jnp.full((pad,), N, jnp.int32)])
    rel_p = jnp.concatenate([rel, jnp.zeros((pad,), jnp.int32)])
    seg = dst_p * R + rel_p
    gidx = rel_p * N + src_p
    dst_p = jnp.minimum(dst_p, NACC - 1)

    scale32 = _sc_prep(seg.reshape(NW, CPW, K))
    gidx32 = gidx.reshape(NW, CPW, K)
    dst32 = dst_p.reshape(NW, CPW, K)

    h1 = _layer(h, bases1, comp1, root1, bias1, gidx32, dst32, scale32, True)
    h2 = _layer(h1, bases2, comp2, root2, bias2, gidx32, dst32, scale32, False)
    return h2.reshape(1, N, D)
